# Initial kernel scaffold; baseline (speedup 1.0000x reference)
import functools

import jax
import jax.numpy as jnp
from jax import lax
from jax.experimental import pallas as pl
from jax.experimental.pallas import tpu as pltpu
from jax.experimental.pallas import tpu_sc as plsc

N = 10000
E = 160000
NB4 = 4

NC, NS, NW = 2, 16, 32
G = 128
EPT = 5120
NGRP = EPT // G
E_PAD = NW * EPT
ACC_ROWS = 10016
STRIPE = ACC_ROWS // NS
DUMMY = 10008

R = 1000
NBLK = N // R


@functools.cache
def _make_segsum(C):
    Cc = min(C, 128)
    nch = C // Cc
    mesh = plsc.VectorSubcoreMesh(
        core_axis_name="c", subcore_axis_name="s", num_cores=NC, num_subcores=NS)

    @functools.partial(
        pl.kernel,
        out_type=jax.ShapeDtypeStruct((NC, nch, ACC_ROWS, Cc), jnp.float32),
        mesh=mesh,
        scratch_types=[
            pltpu.VMEM((NGRP, G), jnp.int32),
            pltpu.VMEM((NGRP, G), jnp.int32),
            pltpu.VMEM((2, G, Cc), jnp.float32),
            pltpu.VMEM_SHARED((ACC_ROWS, Cc), jnp.float32),
            pltpu.SemaphoreType.DMA,
        ],
    )
    def segsum(src_hbm, dst_hbm, g_hbm, zeros_hbm, out_hbm,
               src_v, dst_v, rows_v, acc_sh, sem_g):
        c = lax.axis_index("c")
        s = lax.axis_index("s")
        wid = s * NC + c
        row0 = s * STRIPE
        pltpu.sync_copy(dst_hbm.at[wid], dst_v)
        for ch in range(nch):
            pltpu.sync_copy(zeros_hbm, acc_sh.at[pl.ds(row0, STRIPE)])
            pltpu.sync_copy(src_hbm.at[ch].at[wid], src_v)
            plsc.subcore_barrier()
            pltpu.async_copy(g_hbm.at[src_v.at[0]], rows_v.at[0], sem_g)

            def body(j, _):
                buf = j % 2
                pltpu.make_async_copy(
                    g_hbm.at[src_v.at[j]], rows_v.at[buf], sem_g).wait()

                @pl.when(j < NGRP - 1)
                def _():
                    pltpu.async_copy(
                        g_hbm.at[src_v.at[j + 1]], rows_v.at[(j + 1) % 2],
                        sem_g)

                pltpu.sync_copy(rows_v.at[buf], acc_sh.at[dst_v.at[j]],
                                add=True)
                return 0

            lax.fori_loop(0, NGRP, body, 0)
            plsc.subcore_barrier()
            pltpu.sync_copy(acc_sh.at[pl.ds(row0, STRIPE)],
                            out_hbm.at[c].at[ch].at[pl.ds(row0, STRIPE)])

    return segsum, nch, Cc


def _segsum(g, C, src_scaled, dstp, zeros):
    k, nch, Cc = _make_segsum(C)
    g2 = g.reshape(N * nch, Cc)
    return k(src_scaled, dstp, g2, zeros)


def _mm(h, W, b):
    Cin, Cout = W.shape

    def body(h_ref, w_ref, b_ref, o_ref):
        o_ref[...] = jnp.dot(h_ref[...], w_ref[...],
                             preferred_element_type=jnp.float32) + b_ref[...]

    return pl.pallas_call(
        body,
        grid=(NBLK,),
        in_specs=[
            pl.BlockSpec((R, Cin), lambda i: (i, 0)),
            pl.BlockSpec((Cin, Cout), lambda i: (0, 0)),
            pl.BlockSpec((1, Cout), lambda i: (0, 0)),
        ],
        out_specs=pl.BlockSpec((R, Cout), lambda i: (i, 0)),
        out_shape=jax.ShapeDtypeStruct((N, Cout), jnp.float32),
    )(h, W, b.reshape(1, Cout))


def _p_spec(nch, Cc):
    return pl.BlockSpec((NC, nch, R, Cc), lambda i: (0, 0, i, 0))


def _assemble_agg(p_ref, nch):
    parts = [p_ref[0, ch] + p_ref[1, ch] for ch in range(nch)]
    return jnp.concatenate(parts, axis=1) if nch > 1 else parts[0]


def _stats_update(i, t, s1_ref, s2_ref):
    @pl.when(i == 0)
    def _():
        s1_ref[...] = jnp.zeros_like(s1_ref)
        s2_ref[...] = jnp.zeros_like(s2_ref)

    C = t.shape[1]
    t3 = t.reshape(R // 8, 8, C)
    s1_ref[...] += jnp.sum(t3, axis=0)
    s2_ref[...] += jnp.sum(t3 * t3, axis=0)


def _add_stats(h2, p, C):
    nch, Cc = p.shape[1], p.shape[3]

    def body(h2_ref, p_ref, t_ref, s1_ref, s2_ref):
        t = h2_ref[...] + _assemble_agg(p_ref, nch)
        t_ref[...] = t
        _stats_update(pl.program_id(0), t, s1_ref, s2_ref)

    return pl.pallas_call(
        body,
        grid=(NBLK,),
        in_specs=[pl.BlockSpec((R, C), lambda i: (i, 0)), _p_spec(nch, Cc)],
        out_specs=(
            pl.BlockSpec((R, C), lambda i: (i, 0)),
            pl.BlockSpec((8, C), lambda i: (0, 0)),
            pl.BlockSpec((8, C), lambda i: (0, 0)),
        ),
        out_shape=(
            jax.ShapeDtypeStruct((N, C), jnp.float32),
            jax.ShapeDtypeStruct((8, C), jnp.float32),
            jax.ShapeDtypeStruct((8, C), jnp.float32),
        ),
    )(h2, p)


def _mm_stats(h, p, W, b):
    Cin, Cout = W.shape
    nch, Cc = p.shape[1], p.shape[3]

    def body(h_ref, p_ref, w_ref, b_ref, t_ref, s1_ref, s2_ref):
        u = h_ref[...] + _assemble_agg(p_ref, nch)
        t = jnp.dot(u, w_ref[...],
                    preferred_element_type=jnp.float32) + b_ref[...]
        t_ref[...] = t
        _stats_update(pl.program_id(0), t, s1_ref, s2_ref)

    return pl.pallas_call(
        body,
        grid=(NBLK,),
        in_specs=[
            pl.BlockSpec((R, Cin), lambda i: (i, 0)),
            _p_spec(nch, Cc),
            pl.BlockSpec((Cin, Cout), lambda i: (0, 0)),
            pl.BlockSpec((1, Cout), lambda i: (0, 0)),
        ],
        out_specs=(
            pl.BlockSpec((R, Cout), lambda i: (i, 0)),
            pl.BlockSpec((8, Cout), lambda i: (0, 0)),
            pl.BlockSpec((8, Cout), lambda i: (0, 0)),
        ),
        out_shape=(
            jax.ShapeDtypeStruct((N, Cout), jnp.float32),
            jax.ShapeDtypeStruct((8, Cout), jnp.float32),
            jax.ShapeDtypeStruct((8, Cout), jnp.float32),
        ),
    )(h, p, W, b.reshape(1, Cout))


def _norm_elu(t, s1, s2, C):
    def body(t_ref, s1_ref, s2_ref, o_ref):
        su = jnp.sum(s1_ref[...], axis=0, keepdims=True)
        sq = jnp.sum(s2_ref[...], axis=0, keepdims=True)
        mean = su / N
        var = sq / N - mean * mean
        inv = lax.rsqrt(var + 1e-5)
        xn = (t_ref[...] - mean) * inv
        o_ref[...] = jnp.where(xn > 0, xn, jnp.expm1(xn))

    return pl.pallas_call(
        body,
        grid=(NBLK,),
        in_specs=[
            pl.BlockSpec((R, C), lambda i: (i, 0)),
            pl.BlockSpec((8, C), lambda i: (0, 0)),
            pl.BlockSpec((8, C), lambda i: (0, 0)),
        ],
        out_specs=pl.BlockSpec((R, C), lambda i: (i, 0)),
        out_shape=jax.ShapeDtypeStruct((N, C), jnp.float32),
    )(t, s1, s2)


def _pool(h, ids2d):
    C = h.shape[1]

    def body(h_ref, id_ref, s_ref):
        onehot = (id_ref[...] == jax.lax.broadcasted_iota(
            jnp.int32, (1, NB4), 1)).astype(jnp.float32)
        sums = lax.dot_general(onehot, h_ref[...], (((0,), (0,)), ((), ())),
                               preferred_element_type=jnp.float32)
        cnts = jnp.sum(onehot, axis=0)[:, None]

        @pl.when(pl.program_id(0) == 0)
        def _():
            s_ref[...] = jnp.zeros_like(s_ref)

        s_ref[0:4, :] += sums
        s_ref[4:8, :] += jnp.broadcast_to(cnts, (NB4, C))

    return pl.pallas_call(
        body,
        grid=(NBLK,),
        in_specs=[
            pl.BlockSpec((R, C), lambda i: (i, 0)),
            pl.BlockSpec((R, 1), lambda i: (i, 0)),
        ],
        out_specs=pl.BlockSpec((8, C), lambda i: (0, 0)),
        out_shape=jax.ShapeDtypeStruct((8, C), jnp.float32),
    )(h, ids2d)


def _latent(s, emb_W, emb_b, mu_W, mu_b, lv_W, lv_b, dec_W0):
    def body(s_ref, ew_ref, eb_ref, mw_ref, mb_ref, lw_ref, lb_ref, dw_ref,
             mu_ref, lv_ref, zw_ref):
        pooled = s_ref[0:4, :] / jnp.maximum(s_ref[4:8, :], 1.0)
        emb = jnp.dot(pooled, ew_ref[...],
                      preferred_element_type=jnp.float32) + eb_ref[...]
        mu = jnp.dot(emb, mw_ref[...],
                     preferred_element_type=jnp.float32) + mb_ref[...]
        lv = jnp.dot(emb, lw_ref[...],
                     preferred_element_type=jnp.float32) + lb_ref[...]
        mu_ref[...] = mu
        lv_ref[...] = lv
        zw_ref[...] = jnp.dot(mu, dw_ref[...],
                              preferred_element_type=jnp.float32)

    D = emb_W.shape[0]
    Dout = dec_W0.shape[1]
    return pl.pallas_call(
        body,
        out_shape=(
            jax.ShapeDtypeStruct((NB4, D), jnp.float32),
            jax.ShapeDtypeStruct((NB4, D), jnp.float32),
            jax.ShapeDtypeStruct((NB4, Dout), jnp.float32),
        ),
    )(s, emb_W, emb_b.reshape(1, D), mu_W, mu_b.reshape(1, D),
      lv_W, lv_b.reshape(1, D), dec_W0)


def _bcast_add(ids2d, zW, b):
    C = zW.shape[1]

    def body(id_ref, z_ref, b_ref, o_ref):
        onehot = (id_ref[...] == jax.lax.broadcasted_iota(
            jnp.int32, (1, NB4), 1)).astype(jnp.float32)
        o_ref[...] = jnp.dot(onehot, z_ref[...],
                             preferred_element_type=jnp.float32) + b_ref[...]

    return pl.pallas_call(
        body,
        grid=(NBLK,),
        in_specs=[
            pl.BlockSpec((R, 1), lambda i: (i, 0)),
            pl.BlockSpec((NB4, C), lambda i: (0, 0)),
            pl.BlockSpec((1, C), lambda i: (0, 0)),
        ],
        out_specs=pl.BlockSpec((R, C), lambda i: (i, 0)),
        out_shape=jax.ShapeDtypeStruct((N, C), jnp.float32),
    )(ids2d, zW, b.reshape(1, C))


def kernel(x, edge_index, batch_ids, gt_target, enc_W, enc_b, emb_W, emb_b,
           mu_W, mu_b, lv_W, lv_b, dec_W, dec_b, out_W, out_b):
    src = edge_index[0]
    dst = edge_index[1]

    srcp = jnp.zeros((E_PAD,), jnp.int32).at[:E].set(src)
    dstp = (jnp.full((E_PAD,), DUMMY, jnp.int32).at[:E].set(dst)
            .reshape(NW, NGRP, G))
    src_scaled = {}
    for nch in (1, 2, 4):
        src_scaled[nch] = (
            srcp[None, :] * nch
            + jnp.arange(nch, dtype=jnp.int32)[:, None]
        ).reshape(nch, NW, NGRP, G)
    zeros = {cc: jnp.zeros((STRIPE, cc), jnp.float32)
             for cc in (16, 32, 64, 128)}

    def seg(g, C):
        nch = max(1, C // 128)
        cc = min(C, 128)
        return _segsum(g, C, src_scaled[nch], dstp, zeros[cc])

    ids2d = batch_ids.reshape(N, 1)

    h = x
    for W, b in zip(enc_W, enc_b):
        cin, cout = W.shape
        if cout <= cin:
            h2 = _mm(h, W, b)
            p = seg(h2, cout)
            t, s1, s2 = _add_stats(h2, p, cout)
        else:
            p = seg(h, cin)
            t, s1, s2 = _mm_stats(h, p, W, b)
        h = _norm_elu(t, s1, s2, cout)

    s = _pool(h, ids2d)
    means, log_vars, zW = _latent(s, emb_W, emb_b, mu_W, mu_b, lv_W, lv_b,
                                  dec_W[0])
    zs = means

    h2 = _bcast_add(ids2d, zW, dec_b[0])
    c0 = dec_W[0].shape[1]
    p = seg(h2, c0)
    t, s1, s2 = _add_stats(h2, p, c0)
    h = _norm_elu(t, s1, s2, c0)

    for W, b in zip(dec_W[1:], dec_b[1:]):
        cout = W.shape[1]
        h2 = _mm(h, W, b)
        p = seg(h2, cout)
        t, s1, s2 = _add_stats(h2, p, cout)
        h = _norm_elu(t, s1, s2, cout)

    out_Wp = jnp.pad(out_W, ((0, 0), (0, 127)))
    out_bp = jnp.pad(out_b, (0, 127))
    sout_full = _mm(h, out_Wp, out_bp)
    sout = sout_full[:, :1]

    return sout, means, log_vars, zs


# speedup vs baseline: 2.2689x; 2.2689x over previous
import functools

import jax
import jax.numpy as jnp
from jax import lax
from jax.experimental import pallas as pl
from jax.experimental.pallas import tpu as pltpu
from jax.experimental.pallas import tpu_sc as plsc

N = 10000
E = 160000
NB4 = 4

NC, NS, NW = 2, 16, 32
G = 128
EPT = 5120
NGRP = EPT // G
E_PAD = NW * EPT
ACC_ROWS = 10112
STRIPE = ACC_ROWS // NS
DUMMY = 10008

R = 1000
NBLK = N // R


@functools.cache
def _make_segsum(C):
    Cc = min(C, 128)
    nch = C // Cc
    mesh = plsc.VectorSubcoreMesh(
        core_axis_name="c", subcore_axis_name="s", num_cores=NC, num_subcores=NS)

    @functools.partial(
        pl.kernel,
        out_type=jax.ShapeDtypeStruct((NC, nch, ACC_ROWS, Cc), jnp.float32),
        mesh=mesh,
        scratch_types=[
            pltpu.VMEM((NGRP, G), jnp.int32),
            pltpu.VMEM((NGRP, G), jnp.int32),
            pltpu.VMEM((2, G, Cc), jnp.float32),
            pltpu.VMEM_SHARED((ACC_ROWS, Cc), jnp.float32),
            pltpu.SemaphoreType.DMA,
        ],
        compiler_params=pltpu.CompilerParams(use_tc_tiling_on_sc=False),
    )
    def segsum(src_hbm, dst_hbm, g_hbm, zeros_hbm, out_hbm,
               src_v, dst_v, rows_v, acc_sh, sem_g):
        c = lax.axis_index("c")
        s = lax.axis_index("s")
        wid = c * NS + s
        row0 = s * STRIPE
        pltpu.sync_copy(dst_hbm.at[wid], dst_v)
        for ch in range(nch):
            pltpu.sync_copy(zeros_hbm, acc_sh.at[pl.ds(row0, STRIPE)])
            pltpu.sync_copy(src_hbm.at[ch].at[wid], src_v)
            plsc.subcore_barrier()
            pltpu.async_copy(g_hbm.at[src_v.at[0]], rows_v.at[0], sem_g)

            def body(j, _):
                buf = j % 2
                pltpu.make_async_copy(
                    g_hbm.at[src_v.at[j]], rows_v.at[buf], sem_g).wait()

                @pl.when(j < NGRP - 1)
                def _():
                    pltpu.async_copy(
                        g_hbm.at[src_v.at[j + 1]], rows_v.at[(j + 1) % 2],
                        sem_g)

                pltpu.sync_copy(rows_v.at[buf], acc_sh.at[dst_v.at[j]],
                                add=True)
                return 0

            lax.fori_loop(0, NGRP, body, 0)
            plsc.subcore_barrier()
            pltpu.sync_copy(acc_sh.at[pl.ds(row0, STRIPE)],
                            out_hbm.at[c].at[ch].at[pl.ds(row0, STRIPE)])

    return segsum, nch, Cc


def _segsum(g, C, src_scaled, dstp, zeros):
    k, nch, Cc = _make_segsum(C)
    g2 = g.reshape(N * nch, Cc)
    return k(src_scaled, dstp, g2, zeros)


def _bdot(a, b):
    return jnp.dot(a.astype(jnp.bfloat16), b.astype(jnp.bfloat16),
                   preferred_element_type=jnp.float32)


def _mm(h, W, b):
    Cin, Cout = W.shape

    def body(h_ref, w_ref, b_ref, o_ref):
        o_ref[...] = _bdot(h_ref[...], w_ref[...]) + b_ref[...]

    return pl.pallas_call(
        body,
        grid=(NBLK,),
        in_specs=[
            pl.BlockSpec((R, Cin), lambda i: (i, 0)),
            pl.BlockSpec((Cin, Cout), lambda i: (0, 0)),
            pl.BlockSpec((1, Cout), lambda i: (0, 0)),
        ],
        out_specs=pl.BlockSpec((R, Cout), lambda i: (i, 0)),
        out_shape=jax.ShapeDtypeStruct((N, Cout), jnp.float32),
    )(h, W, b.reshape(1, Cout))


def _p_spec(nch, Cc):
    return pl.BlockSpec((NC, nch, R, Cc), lambda i: (0, 0, i, 0))


def _assemble_agg(p_ref, nch):
    parts = [p_ref[0, ch] + p_ref[1, ch] for ch in range(nch)]
    return jnp.concatenate(parts, axis=1) if nch > 1 else parts[0]


def _stats_update(i, t, s1_ref, s2_ref):
    @pl.when(i == 0)
    def _():
        s1_ref[...] = jnp.zeros_like(s1_ref)
        s2_ref[...] = jnp.zeros_like(s2_ref)

    C = t.shape[1]
    t3 = t.reshape(R // 8, 8, C)
    s1_ref[...] += jnp.sum(t3, axis=0)
    s2_ref[...] += jnp.sum(t3 * t3, axis=0)


def _add_stats(h2, p, C):
    nch, Cc = p.shape[1], p.shape[3]

    def body(h2_ref, p_ref, t_ref, s1_ref, s2_ref):
        t = h2_ref[...] + _assemble_agg(p_ref, nch)
        t_ref[...] = t
        _stats_update(pl.program_id(0), t, s1_ref, s2_ref)

    return pl.pallas_call(
        body,
        grid=(NBLK,),
        in_specs=[pl.BlockSpec((R, C), lambda i: (i, 0)), _p_spec(nch, Cc)],
        out_specs=(
            pl.BlockSpec((R, C), lambda i: (i, 0)),
            pl.BlockSpec((8, C), lambda i: (0, 0)),
            pl.BlockSpec((8, C), lambda i: (0, 0)),
        ),
        out_shape=(
            jax.ShapeDtypeStruct((N, C), jnp.float32),
            jax.ShapeDtypeStruct((8, C), jnp.float32),
            jax.ShapeDtypeStruct((8, C), jnp.float32),
        ),
    )(h2, p)


def _mm_stats(h, p, W, b):
    Cin, Cout = W.shape
    nch, Cc = p.shape[1], p.shape[3]

    def body(h_ref, p_ref, w_ref, b_ref, t_ref, s1_ref, s2_ref):
        u = h_ref[...] + _assemble_agg(p_ref, nch)
        t = _bdot(u, w_ref[...]) + b_ref[...]
        t_ref[...] = t
        _stats_update(pl.program_id(0), t, s1_ref, s2_ref)

    return pl.pallas_call(
        body,
        grid=(NBLK,),
        in_specs=[
            pl.BlockSpec((R, Cin), lambda i: (i, 0)),
            _p_spec(nch, Cc),
            pl.BlockSpec((Cin, Cout), lambda i: (0, 0)),
            pl.BlockSpec((1, Cout), lambda i: (0, 0)),
        ],
        out_specs=(
            pl.BlockSpec((R, Cout), lambda i: (i, 0)),
            pl.BlockSpec((8, Cout), lambda i: (0, 0)),
            pl.BlockSpec((8, Cout), lambda i: (0, 0)),
        ),
        out_shape=(
            jax.ShapeDtypeStruct((N, Cout), jnp.float32),
            jax.ShapeDtypeStruct((8, Cout), jnp.float32),
            jax.ShapeDtypeStruct((8, Cout), jnp.float32),
        ),
    )(h, p, W, b.reshape(1, Cout))


def _norm_elu(t, s1, s2, C):
    def body(t_ref, s1_ref, s2_ref, o_ref):
        su = jnp.sum(s1_ref[...], axis=0, keepdims=True)
        sq = jnp.sum(s2_ref[...], axis=0, keepdims=True)
        mean = su / N
        var = sq / N - mean * mean
        inv = 1.0 / jnp.sqrt(var + 1e-5)
        xn = (t_ref[...] - mean) * inv
        o_ref[...] = jnp.where(xn > 0, xn, jnp.exp(jnp.minimum(xn, 0.0)) - 1.0)

    return pl.pallas_call(
        body,
        grid=(NBLK,),
        in_specs=[
            pl.BlockSpec((R, C), lambda i: (i, 0)),
            pl.BlockSpec((8, C), lambda i: (0, 0)),
            pl.BlockSpec((8, C), lambda i: (0, 0)),
        ],
        out_specs=pl.BlockSpec((R, C), lambda i: (i, 0)),
        out_shape=jax.ShapeDtypeStruct((N, C), jnp.float32),
    )(t, s1, s2)


def _pool(h, ids2d):
    C = h.shape[1]

    def body(h_ref, id_ref, s_ref):
        onehot = (id_ref[...] == jax.lax.broadcasted_iota(
            jnp.int32, (1, NB4), 1)).astype(jnp.float32)
        sums = lax.dot_general(onehot, h_ref[...], (((0,), (0,)), ((), ())),
                               preferred_element_type=jnp.float32,
                               precision=lax.Precision.HIGHEST)
        cnts = jnp.sum(onehot, axis=0)[:, None]

        @pl.when(pl.program_id(0) == 0)
        def _():
            s_ref[...] = jnp.zeros_like(s_ref)

        s_ref[0:4, :] += sums
        s_ref[4:8, :] += jnp.broadcast_to(cnts, (NB4, C))

    return pl.pallas_call(
        body,
        grid=(NBLK,),
        in_specs=[
            pl.BlockSpec((R, C), lambda i: (i, 0)),
            pl.BlockSpec((R, 1), lambda i: (i, 0)),
        ],
        out_specs=pl.BlockSpec((8, C), lambda i: (0, 0)),
        out_shape=jax.ShapeDtypeStruct((8, C), jnp.float32),
    )(h, ids2d)


def _latent(s, emb_W, emb_b, mu_W, mu_b, lv_W, lv_b):
    def body(s_ref, ew_ref, eb_ref, mw_ref, mb_ref, lw_ref, lb_ref,
             mu_ref, lv_ref):
        pooled = s_ref[0:4, :] / jnp.maximum(s_ref[4:8, :], 1.0)
        emb = _bdot(pooled, ew_ref[...]) + eb_ref[...]
        mu = _bdot(emb, mw_ref[...]) + mb_ref[...]
        lv = _bdot(emb, lw_ref[...]) + lb_ref[...]
        mu_ref[...] = mu
        lv_ref[...] = lv

    D = emb_W.shape[0]
    return pl.pallas_call(
        body,
        out_shape=(
            jax.ShapeDtypeStruct((NB4, D), jnp.float32),
            jax.ShapeDtypeStruct((NB4, D), jnp.float32),
        ),
    )(s, emb_W, emb_b.reshape(1, D), mu_W, mu_b.reshape(1, D),
      lv_W, lv_b.reshape(1, D))


def _bcast(ids2d, z):
    C = z.shape[1]

    def body(id_ref, z_ref, o_ref):
        onehot = (id_ref[...] == jax.lax.broadcasted_iota(
            jnp.int32, (1, NB4), 1)).astype(jnp.float32)
        o_ref[...] = jnp.dot(onehot, z_ref[...],
                             preferred_element_type=jnp.float32,
                             precision=lax.Precision.HIGHEST)

    return pl.pallas_call(
        body,
        grid=(NBLK,),
        in_specs=[
            pl.BlockSpec((R, 1), lambda i: (i, 0)),
            pl.BlockSpec((NB4, C), lambda i: (0, 0)),
        ],
        out_specs=pl.BlockSpec((R, C), lambda i: (i, 0)),
        out_shape=jax.ShapeDtypeStruct((N, C), jnp.float32),
    )(ids2d, z)


def kernel(x, edge_index, batch_ids, gt_target, enc_W, enc_b, emb_W, emb_b,
           mu_W, mu_b, lv_W, lv_b, dec_W, dec_b, out_W, out_b):
    src = edge_index[0]
    dst = edge_index[1]

    perm = jnp.argsort(dst, stable=True)
    src_s = src[perm]
    dst_s = dst[perm]
    srcp = jnp.zeros((E_PAD,), jnp.int32).at[:E].set(src_s)
    dstp = (jnp.full((E_PAD,), DUMMY, jnp.int32).at[:E].set(dst_s)
            .reshape(NW, NGRP, G))
    src_scaled = {}
    for nch in (1, 2, 4):
        src_scaled[nch] = (
            srcp[None, :] * nch
            + jnp.arange(nch, dtype=jnp.int32)[:, None]
        ).reshape(nch, NW, NGRP, G)
    zeros = {cc: jnp.zeros((STRIPE, cc), jnp.float32)
             for cc in (16, 32, 64, 128)}

    def seg(g, C):
        nch = max(1, C // 128)
        cc = min(C, 128)
        return _segsum(g, C, src_scaled[nch], dstp, zeros[cc])

    ids2d = batch_ids.reshape(N, 1)

    h = x
    for W, b in zip(enc_W, enc_b):
        cin, cout = W.shape
        if cout <= cin:
            h2 = _mm(h, W, b)
            p = seg(h2, cout)
            t, s1, s2 = _add_stats(h2, p, cout)
        else:
            p = seg(h, cin)
            t, s1, s2 = _mm_stats(h, p, W, b)
        h = _norm_elu(t, s1, s2, cout)

    s = _pool(h, ids2d)
    means, log_vars = _latent(s, emb_W, emb_b, mu_W, mu_b, lv_W, lv_b)
    zs = means

    h = _bcast(ids2d, means)
    for W, b in zip(dec_W, dec_b):
        cout = W.shape[1]
        h2 = _mm(h, W, b)
        p = seg(h2, cout)
        t, s1, s2 = _add_stats(h2, p, cout)
        h = _norm_elu(t, s1, s2, cout)

    out_Wp = jnp.pad(out_W, ((0, 0), (0, 127)))
    out_bp = jnp.pad(out_b, (0, 127))
    sout_full = _mm(h, out_Wp, out_bp)
    sout = sout_full[:, :1]

    return sout, means, log_vars, zs
